# baseline (speedup 1.0000x reference)
import functools

import jax
import jax.numpy as jnp
from jax import lax
from jax.experimental import pallas as pl
from jax.experimental.pallas import tpu as pltpu
from jax.experimental.pallas import tpu_sc as plsc

_W = 128
_SUB = 2
_G = _W // _SUB
_D = 12


@functools.lru_cache(maxsize=None)
def _build(batch, seq, vocab, embed):
    info = plsc.get_sparse_core_info()
    nc, ns = info.num_cores, info.num_subcores
    nw = nc * ns
    assert batch % (nw * _W) == 0
    num_idx = batch * seq
    per_w = num_idx // nw
    nchunk = seq * _SUB
    lag = _D // 2
    head = lag
    mid = ((nchunk - head - lag) // _D) * _D
    assert mid > 0

    mesh = plsc.VectorSubcoreMesh(core_axis_name="c", subcore_axis_name="s")

    @functools.partial(
        pl.kernel,
        out_type=jax.ShapeDtypeStruct((num_idx, embed), jnp.float32),
        mesh=mesh,
        scratch_types=[
            pltpu.VMEM((per_w,), jnp.int32),
        ]
        + [pltpu.VMEM((_G, embed), jnp.float32) for _ in range(_D)]
        + [pltpu.SemaphoreType.DMA for _ in range(2 * _D)],
    )
    def gather_kernel(idx_hbm, table_hbm, out_hbm, idx_v, *rest):
        bufs = rest[:_D]
        gsem = rest[_D : 2 * _D]
        ssem = rest[2 * _D :]

        wid = lax.axis_index("s") * nc + lax.axis_index("c")
        pltpu.sync_copy(idx_hbm.at[pl.ds(wid * per_w, per_w)], idx_v)
        obase = wid * _W

        def gather(c, b):
            pltpu.async_copy(
                table_hbm.at[idx_v.at[pl.ds(c * _G, _G)]], bufs[b], gsem[b]
            )

        def store(c, b):
            t = c // _SUB
            h = c % _SUB
            pltpu.async_copy(
                bufs[b],
                out_hbm.at[pl.ds(t * batch + obase + h * _G, _G)],
                ssem[b],
            )

        def wait_gather(b):
            pltpu.make_async_copy(
                out_hbm.at[pl.ds(obase, _G)], bufs[b], gsem[b]
            ).wait()

        def wait_store(b):
            pltpu.make_async_copy(
                bufs[b], out_hbm.at[pl.ds(obase, _G)], ssem[b]
            ).wait()

        def body(c, bc, with_gather, with_wait_store):
            if with_gather:
                bg = (bc + lag) % _D
                if with_wait_store:
                    wait_store(bg)
                gather(c + lag, bg)
            wait_gather(bc)
            store(c, bc)

        for c in range(lag):
            gather(c, c % _D)
        for c in range(head):
            body(c, c % _D, True, c + lag >= _D)

        @pl.loop(head, head + mid, step=_D)
        def _(g):
            for j in range(_D):
                body(g + j, (head + j) % _D, True, True)

        for c in range(head + mid, nchunk):
            if c + lag < nchunk:
                body(c, c % _D, True, True)
            else:
                body(c, c % _D, False, False)
        for b in range(_D):
            wait_store(b)

    return gather_kernel


def kernel(data, ivectors):
    b, t = data.shape
    vocab, embed = ivectors.shape
    nw = 32
    idx = (
        data.astype(jnp.int32)
        .reshape(nw, b // nw, t)
        .transpose(0, 2, 1)
        .reshape(-1)
    )
    out = _build(b, t, vocab, embed)(idx, ivectors)
    return out.reshape(t, b, embed).transpose(1, 0, 2)


# speedup vs baseline: 1.0013x; 1.0013x over previous
import functools

import jax
import jax.numpy as jnp
from jax import lax
from jax.experimental import pallas as pl
from jax.experimental.pallas import tpu as pltpu
from jax.experimental.pallas import tpu_sc as plsc

_G = 128
_D = 6


@functools.lru_cache(maxsize=None)
def _build(batch, seq, vocab, embed):
    info = plsc.get_sparse_core_info()
    nc, ns = info.num_cores, info.num_subcores
    nw = nc * ns
    assert batch % (nw * _G) == 0
    num_idx = batch * seq
    per_w = num_idx // nw
    nchunk = seq
    lag = _D // 2
    head = lag
    mid = ((nchunk - head - lag) // _D) * _D
    assert mid > 0

    mesh = plsc.VectorSubcoreMesh(core_axis_name="c", subcore_axis_name="s")

    @functools.partial(
        pl.kernel,
        out_type=jax.ShapeDtypeStruct((num_idx, embed), jnp.float32),
        mesh=mesh,
        compiler_params=pltpu.CompilerParams(skip_device_barrier=True),
        scratch_types=[
            pltpu.VMEM((per_w,), jnp.int32),
        ]
        + [pltpu.VMEM((_G, embed), jnp.float32) for _ in range(_D)]
        + [pltpu.SemaphoreType.DMA for _ in range(2 * _D)],
    )
    def gather_kernel(idx_hbm, table_hbm, out_hbm, idx_v, *rest):
        bufs = rest[:_D]
        gsem = rest[_D : 2 * _D]
        ssem = rest[2 * _D :]

        wid = lax.axis_index("s") * nc + lax.axis_index("c")
        pltpu.sync_copy(idx_hbm.at[pl.ds(wid * per_w, per_w)], idx_v)
        obase = wid * _G

        def gather(c, b):
            pltpu.async_copy(
                table_hbm.at[idx_v.at[pl.ds(c * _G, _G)]], bufs[b], gsem[b]
            )

        def store(c, b):
            pltpu.async_copy(
                bufs[b], out_hbm.at[pl.ds(c * batch + obase, _G)], ssem[b]
            )

        def wait_gather(b):
            pltpu.make_async_copy(
                out_hbm.at[pl.ds(obase, _G)], bufs[b], gsem[b]
            ).wait()

        def wait_store(b):
            pltpu.make_async_copy(
                bufs[b], out_hbm.at[pl.ds(obase, _G)], ssem[b]
            ).wait()

        def body(c, bc, with_gather, with_wait_store):
            if with_gather:
                bg = (bc + lag) % _D
                if with_wait_store:
                    wait_store(bg)
                gather(c + lag, bg)
            wait_gather(bc)
            store(c, bc)

        for c in range(lag):
            gather(c, c % _D)
        for c in range(head):
            body(c, c % _D, True, c + lag >= _D)

        @pl.loop(head, head + mid, step=_D)
        def _(g):
            for j in range(_D):
                body(g + j, (head + j) % _D, True, True)

        for c in range(head + mid, nchunk):
            if c + lag < nchunk:
                body(c, c % _D, True, True)
            else:
                body(c, c % _D, False, False)
        for b in range(_D):
            wait_store(b)

    return gather_kernel


def kernel(data, ivectors):
    b, t = data.shape
    vocab, embed = ivectors.shape
    nw = 32
    idx = (
        data.astype(jnp.int32)
        .reshape(nw, b // nw, t)
        .transpose(0, 2, 1)
        .reshape(-1)
    )
    out = _build(b, t, vocab, embed)(idx, ivectors)
    return out.reshape(t, b, embed).transpose(1, 0, 2)


# speedup vs baseline: 1.0080x; 1.0066x over previous
import functools

import jax
import jax.numpy as jnp
from jax import lax
from jax.experimental import pallas as pl
from jax.experimental.pallas import tpu as pltpu
from jax.experimental.pallas import tpu_sc as plsc

_G = 128
_D = 7
_LAG = 4


@functools.lru_cache(maxsize=None)
def _build(batch, seq, vocab, embed):
    info = plsc.get_sparse_core_info()
    nc, ns = info.num_cores, info.num_subcores
    nw = nc * ns
    assert batch % (nw * _G) == 0
    num_idx = batch * seq
    per_w = num_idx // nw
    nchunk = seq
    lag = _LAG
    assert 2 * lag >= _D
    head = lag
    mid = ((nchunk - head - lag) // _D) * _D
    assert mid > 0

    mesh = plsc.VectorSubcoreMesh(core_axis_name="c", subcore_axis_name="s")

    @functools.partial(
        pl.kernel,
        out_type=jax.ShapeDtypeStruct((num_idx, embed), jnp.float32),
        mesh=mesh,
        scratch_types=[
            pltpu.VMEM((per_w,), jnp.int32),
        ]
        + [pltpu.VMEM((_G, embed), jnp.float32) for _ in range(_D)]
        + [pltpu.SemaphoreType.DMA for _ in range(2 * _D)],
    )
    def gather_kernel(idx_hbm, table_hbm, out_hbm, idx_v, *rest):
        bufs = rest[:_D]
        gsem = rest[_D : 2 * _D]
        ssem = rest[2 * _D :]

        wid = lax.axis_index("s") * nc + lax.axis_index("c")
        pltpu.sync_copy(idx_hbm.at[pl.ds(wid * per_w, per_w)], idx_v)
        obase = wid * _G

        def gather(c, b):
            pltpu.async_copy(
                table_hbm.at[idx_v.at[pl.ds(c * _G, _G)]], bufs[b], gsem[b]
            )

        def store(c, b):
            pltpu.async_copy(
                bufs[b], out_hbm.at[pl.ds(c * batch + obase, _G)], ssem[b]
            )

        def wait_gather(b):
            pltpu.make_async_copy(
                out_hbm.at[pl.ds(obase, _G)], bufs[b], gsem[b]
            ).wait()

        def wait_store(b):
            pltpu.make_async_copy(
                bufs[b], out_hbm.at[pl.ds(obase, _G)], ssem[b]
            ).wait()

        def body(c, bc, with_gather, with_wait_store):
            if with_gather:
                bg = (bc + lag) % _D
                if with_wait_store:
                    wait_store(bg)
                gather(c + lag, bg)
            wait_gather(bc)
            store(c, bc)

        for c in range(lag):
            gather(c, c % _D)
        for c in range(head):
            body(c, c % _D, True, c + lag >= _D)

        @pl.loop(head, head + mid, step=_D)
        def _(g):
            for j in range(_D):
                body(g + j, (head + j) % _D, True, True)

        for c in range(head + mid, nchunk):
            if c + lag < nchunk:
                body(c, c % _D, True, True)
            else:
                body(c, c % _D, False, False)
        for b in range(_D):
            wait_store(b)

    return gather_kernel


def kernel(data, ivectors):
    b, t = data.shape
    vocab, embed = ivectors.shape
    nw = 32
    idx = (
        data.astype(jnp.int32)
        .reshape(nw, b // nw, t)
        .transpose(0, 2, 1)
        .reshape(-1)
    )
    out = _build(b, t, vocab, embed)(idx, ivectors)
    return out.reshape(t, b, embed).transpose(1, 0, 2)
